# initial kernel scaffold (unmeasured)
import jax
import jax.numpy as jnp
from jax import lax
from jax.experimental import pallas as pl
from jax.experimental.pallas import tpu as pltpu

B, H, D, BS = 16, 16, 64, 16
NB = 128
PAGES = 128
HD = H * D
NK = PAGES * BS
SCALE = D ** -0.5
NEG = -1e30


def _body(q_ref, k_ref, v_ref, bt_ref, lens_ref, out_ref,
          o_send, o_recv, m_send, m_recv, l_send, l_recv,
          send_sems, recv_sems):
    my_x = lax.axis_index("x")
    my_y = lax.axis_index("y")
    my_z = lax.axis_index("z")
    nbr = (my_x, 1 - my_y, my_z)

    barrier = pltpu.get_barrier_semaphore()
    pl.semaphore_signal(barrier, inc=1, device_id=nbr,
                        device_id_type=pl.DeviceIdType.MESH)
    pl.semaphore_wait(barrier, 1)

    hd_mask = (lax.broadcasted_iota(jnp.int32, (H, HD), 1) // D
               == lax.broadcasted_iota(jnp.int32, (H, HD), 0))
    pk = lax.broadcasted_iota(jnp.int32, (NK, NB), 0) // BS
    slot = lax.broadcasted_iota(jnp.int32, (NK, NB), 1)

    kb = k_ref[...]
    vb = v_ref[...]

    for b in range(B):
        qrow = q_ref[b:b + 1, :]
        qbT = jnp.where(hd_mask, qrow, 0).astype(jnp.bfloat16)
        s = lax.dot_general(kb, qbT, (((1,), (1,)), ((), ())),
                            preferred_element_type=jnp.float32)
        s = s * SCALE

        btrow = bt_ref[b:b + 1, :] - my_y * PAGES
        valid = slot < lens_ref[b]
        match = jnp.logical_and(pk == btrow, valid)
        cntk = jnp.sum(match.astype(jnp.float32), axis=1, keepdims=True)

        smask = jnp.where(cntk > 0, s, NEG)
        m = jnp.max(smask, axis=0, keepdims=True)
        m_safe = jnp.where(m < -1e29, 0.0, m)
        p = jnp.exp(s - m_safe) * cntk
        l = jnp.sum(p, axis=0, keepdims=True)

        r = lax.dot_general(p.astype(jnp.bfloat16), vb,
                            (((0,), (0,)), ((), ())),
                            preferred_element_type=jnp.float32)
        o_flat = jnp.sum(jnp.where(hd_mask, r, 0.0), axis=0, keepdims=True)

        o_send[b:b + 1, :] = o_flat
        m_send[b:b + 1, :] = m
        l_send[b:b + 1, :] = l

    rdmas = []
    for i, (src, dst) in enumerate(
            [(o_send, o_recv), (m_send, m_recv), (l_send, l_recv)]):
        rdma = pltpu.make_async_remote_copy(
            src_ref=src, dst_ref=dst,
            send_sem=send_sems.at[i], recv_sem=recv_sems.at[i],
            device_id=nbr, device_id_type=pl.DeviceIdType.MESH)
        rdma.start()
        rdmas.append(rdma)
    for rdma in rdmas:
        rdma.wait()

    m_loc = m_send[...]
    m_rem = m_recv[...]
    mt = jnp.maximum(m_loc, m_rem)
    a = jnp.exp(m_loc - mt)
    c = jnp.exp(m_rem - mt)
    lt = a * l_send[...] + c * l_recv[...]
    hd_maskf = hd_mask.astype(jnp.float32)
    a_e = jnp.dot(a, hd_maskf, preferred_element_type=jnp.float32)
    c_e = jnp.dot(c, hd_maskf, preferred_element_type=jnp.float32)
    l_e = jnp.dot(lt, hd_maskf, preferred_element_type=jnp.float32)
    out_ref[...] = (a_e * o_send[...] + c_e * o_recv[...]) / l_e


def kernel(Q, K, V, bt, lens):
    Qr = Q.reshape(B, HD).astype(jnp.bfloat16)
    Kr = K.astype(jnp.bfloat16).reshape(NK, HD)
    Vr = V.astype(jnp.bfloat16).reshape(NK, HD)

    out = pl.pallas_call(
        _body,
        out_shape=jax.ShapeDtypeStruct((B, HD), jnp.float32),
        in_specs=[
            pl.BlockSpec(memory_space=pltpu.VMEM),
            pl.BlockSpec(memory_space=pltpu.VMEM),
            pl.BlockSpec(memory_space=pltpu.VMEM),
            pl.BlockSpec(memory_space=pltpu.VMEM),
            pl.BlockSpec(memory_space=pltpu.SMEM),
        ],
        out_specs=pl.BlockSpec(memory_space=pltpu.VMEM),
        scratch_shapes=[
            pltpu.VMEM((B, HD), jnp.float32),
            pltpu.VMEM((B, HD), jnp.float32),
            pltpu.VMEM((B, H), jnp.float32),
            pltpu.VMEM((B, H), jnp.float32),
            pltpu.VMEM((B, H), jnp.float32),
            pltpu.VMEM((B, H), jnp.float32),
            pltpu.SemaphoreType.DMA((3,)),
            pltpu.SemaphoreType.DMA((3,)),
        ],
        compiler_params=pltpu.CompilerParams(collective_id=0),
    )(Qr, Kr, Vr, bt, lens)
    return out.reshape(B, 1, H, D)


# baseline (device time: 104578 ns/iter reference)
import jax
import jax.numpy as jnp
from jax import lax
from jax.experimental import pallas as pl
from jax.experimental.pallas import tpu as pltpu

B, H, D, BS = 16, 16, 64, 16
NB = 128
PAGES = 128
HD = H * D
NK = PAGES * BS
SCALE = D ** -0.5
NEG = -1e30


def _body(q_ref, k_ref, v_ref, bt_ref, lens_ref, out_ref,
          o_send, o_recv, m_send, m_recv, l_send, l_recv,
          send_sems, recv_sems):
    my_x = lax.axis_index("x")
    my_y = lax.axis_index("y")
    my_z = lax.axis_index("z")
    nbr = (my_x, 1 - my_y, my_z)

    barrier = pltpu.get_barrier_semaphore()
    pl.semaphore_signal(barrier, inc=1, device_id=nbr,
                        device_id_type=pl.DeviceIdType.MESH)
    pl.semaphore_wait(barrier, 1)

    def _hd_mask(dtype):
        return (lax.broadcasted_iota(jnp.int32, (H, HD), 1) // D
                == lax.broadcasted_iota(jnp.int32, (H, HD), 0)).astype(dtype)

    hd_mask_bf = _hd_mask(jnp.bfloat16)
    hd_mask_f32 = _hd_mask(jnp.float32)
    pk = lax.broadcasted_iota(jnp.int32, (NK, NB), 0) // BS
    slot = lax.broadcasted_iota(jnp.int32, (NK, NB), 1)

    kb = k_ref[...]
    vb = v_ref[...]

    for b in range(B):
        qrow = q_ref[b:b + 1, :]
        qbT = qrow * hd_mask_bf
        s = lax.dot_general(kb, qbT, (((1,), (1,)), ((), ())),
                            preferred_element_type=jnp.float32)
        s = s * SCALE

        btrow = bt_ref[b:b + 1, :] - my_y * PAGES
        valid = slot < lens_ref[b]
        match = jnp.logical_and(pk == btrow, valid)
        cntk = jnp.sum(match.astype(jnp.float32), axis=1, keepdims=True)

        smask = jnp.where(cntk > 0, s, NEG)
        m = jnp.max(smask, axis=0, keepdims=True)
        m_safe = jnp.where(m < -1e29, 0.0, m)
        p = jnp.exp(s - m_safe) * cntk
        l = jnp.sum(p, axis=0, keepdims=True)

        r = lax.dot_general(p.astype(jnp.bfloat16), vb,
                            (((0,), (0,)), ((), ())),
                            preferred_element_type=jnp.float32)
        o_flat = jnp.sum(r * hd_mask_f32, axis=0, keepdims=True)

        o_send[b:b + 1, :] = o_flat
        m_send[b:b + 1, :] = m
        l_send[b:b + 1, :] = l

    rdmas = []
    for i, (src, dst) in enumerate(
            [(o_send, o_recv), (m_send, m_recv), (l_send, l_recv)]):
        rdma = pltpu.make_async_remote_copy(
            src_ref=src, dst_ref=dst,
            send_sem=send_sems.at[i], recv_sem=recv_sems.at[i],
            device_id=nbr, device_id_type=pl.DeviceIdType.MESH)
        rdma.start()
        rdmas.append(rdma)
    for rdma in rdmas:
        rdma.wait()

    m_loc = m_send[...]
    m_rem = m_recv[...]
    mt = jnp.maximum(m_loc, m_rem)
    a = jnp.exp(m_loc - mt)
    c = jnp.exp(m_rem - mt)
    lt = a * l_send[...] + c * l_recv[...]
    a_e = jnp.dot(a, hd_mask_f32, preferred_element_type=jnp.float32)
    c_e = jnp.dot(c, hd_mask_f32, preferred_element_type=jnp.float32)
    l_e = jnp.dot(lt, hd_mask_f32, preferred_element_type=jnp.float32)
    out_ref[...] = (a_e * o_send[...] + c_e * o_recv[...]) / l_e


def kernel(Q, K, V, bt, lens):
    Qr = Q.reshape(B, HD).astype(jnp.bfloat16)
    Kr = K.astype(jnp.bfloat16).reshape(NK, HD)
    Vr = V.astype(jnp.bfloat16).reshape(NK, HD)

    out = pl.pallas_call(
        _body,
        out_shape=jax.ShapeDtypeStruct((B, HD), jnp.float32),
        in_specs=[
            pl.BlockSpec(memory_space=pltpu.VMEM),
            pl.BlockSpec(memory_space=pltpu.VMEM),
            pl.BlockSpec(memory_space=pltpu.VMEM),
            pl.BlockSpec(memory_space=pltpu.VMEM),
            pl.BlockSpec(memory_space=pltpu.SMEM),
        ],
        out_specs=pl.BlockSpec(memory_space=pltpu.VMEM),
        scratch_shapes=[
            pltpu.VMEM((B, HD), jnp.float32),
            pltpu.VMEM((B, HD), jnp.float32),
            pltpu.VMEM((B, H), jnp.float32),
            pltpu.VMEM((B, H), jnp.float32),
            pltpu.VMEM((B, H), jnp.float32),
            pltpu.VMEM((B, H), jnp.float32),
            pltpu.SemaphoreType.DMA((3,)),
            pltpu.SemaphoreType.DMA((3,)),
        ],
        compiler_params=pltpu.CompilerParams(collective_id=0),
    )(Qr, Kr, Vr, bt, lens)
    return out.reshape(B, 1, H, D)


# device time: 53887 ns/iter; 1.9407x vs baseline; 1.9407x over previous
import jax
import jax.numpy as jnp
from jax import lax
from jax.experimental import pallas as pl
from jax.experimental.pallas import tpu as pltpu

B, H, D, BS = 16, 16, 64, 16
NB = 128
PAGES = 128
HD = H * D
NK = PAGES * BS
BH = B * H
SCALE = D ** -0.5
NEG = -1e30


def _iota2(shape, dim):
    return lax.broadcasted_iota(jnp.int32, shape, dim)


def _body(q_ref, k_ref, v_ref, bt_ref, lens_ref, out_ref,
          cnt_all, o_send, o_recv, m_send, m_recv, l_send, l_recv,
          send_sems, recv_sems):
    my_x = lax.axis_index("x")
    my_y = lax.axis_index("y")
    my_z = lax.axis_index("z")
    nbr = (my_x, 1 - my_y, my_z)

    barrier = pltpu.get_barrier_semaphore()
    pl.semaphore_signal(barrier, inc=1, device_id=nbr,
                        device_id_type=pl.DeviceIdType.MESH)
    pl.semaphore_wait(barrier, 1)

    maskB_bf = (_iota2((BH, HD), 1) // D == _iota2((BH, HD), 0) % H
                ).astype(jnp.bfloat16)
    maskB_f32 = (_iota2((BH, HD), 1) // D == _iota2((BH, HD), 0) % H
                 ).astype(jnp.float32)
    erep_bf = (_iota2((BH, B), 0) // H == _iota2((BH, B), 1)
               ).astype(jnp.bfloat16)
    efold_f32 = (_iota2((B, BH), 1) // H == _iota2((B, BH), 0)
                 ).astype(jnp.float32)
    ecol_bf = (_iota2((B, BH), 1) // H == _iota2((B, BH), 0)
               ).astype(jnp.bfloat16)
    pk = _iota2((NK, NB), 0) // BS
    slot = _iota2((1, NB), 1)

    kb = k_ref[...]
    vb = v_ref[...]
    qr = q_ref[...]

    for b in range(B):
        btrow = bt_ref[b:b + 1, :] - my_y * PAGES
        match = jnp.logical_and(pk == btrow, slot < lens_ref[b])
        cnt_all[:, b:b + 1] = jnp.sum(match.astype(jnp.float32),
                                      axis=1, keepdims=True)

    cntcol = lax.dot_general(
        cnt_all[...].astype(jnp.bfloat16), ecol_bf,
        (((1,), (0,)), ((), ())), preferred_element_type=jnp.float32)

    qrep = lax.dot_general(erep_bf, qr, (((1,), (0,)), ((), ())),
                           preferred_element_type=jnp.float32)
    qbT = qrep.astype(jnp.bfloat16) * maskB_bf

    s = lax.dot_general(kb, qbT, (((1,), (1,)), ((), ())),
                        preferred_element_type=jnp.float32) * SCALE

    smask = jnp.where(cntcol > 0, s, NEG)
    m = jnp.max(smask, axis=0, keepdims=True)
    m_safe = jnp.where(m < -1e29, 0.0, m)
    p = jnp.exp(s - m_safe) * cntcol
    l = jnp.sum(p, axis=0, keepdims=True)

    r = lax.dot_general(p.astype(jnp.bfloat16), vb,
                        (((0,), (0,)), ((), ())),
                        preferred_element_type=jnp.float32)
    o_send[...] = jnp.dot(efold_f32, r * maskB_f32,
                          preferred_element_type=jnp.float32)
    m_send[...] = m
    l_send[...] = l

    rdmas = []
    for i, (src, dst) in enumerate(
            [(o_send, o_recv), (m_send, m_recv), (l_send, l_recv)]):
        rdma = pltpu.make_async_remote_copy(
            src_ref=src, dst_ref=dst,
            send_sem=send_sems.at[i], recv_sem=recv_sems.at[i],
            device_id=nbr, device_id_type=pl.DeviceIdType.MESH)
        rdma.start()
        rdmas.append(rdma)
    for rdma in rdmas:
        rdma.wait()

    m_loc = m_send[...]
    m_rem = m_recv[...]
    mt = jnp.maximum(m_loc, m_rem)
    a = jnp.exp(m_loc - mt)
    c = jnp.exp(m_rem - mt)
    lt = a * l_send[...] + c * l_recv[...]
    a_e = jnp.dot(a * efold_f32, maskB_f32, preferred_element_type=jnp.float32)
    c_e = jnp.dot(c * efold_f32, maskB_f32, preferred_element_type=jnp.float32)
    l_e = jnp.dot(lt * efold_f32, maskB_f32, preferred_element_type=jnp.float32)
    out_ref[...] = (a_e * o_send[...] + c_e * o_recv[...]) / l_e


def kernel(Q, K, V, bt, lens):
    Qr = Q.reshape(B, HD).astype(jnp.bfloat16)
    Kr = K.astype(jnp.bfloat16).reshape(NK, HD)
    Vr = V.astype(jnp.bfloat16).reshape(NK, HD)

    out = pl.pallas_call(
        _body,
        out_shape=jax.ShapeDtypeStruct((B, HD), jnp.float32),
        in_specs=[
            pl.BlockSpec(memory_space=pltpu.VMEM),
            pl.BlockSpec(memory_space=pltpu.VMEM),
            pl.BlockSpec(memory_space=pltpu.VMEM),
            pl.BlockSpec(memory_space=pltpu.VMEM),
            pl.BlockSpec(memory_space=pltpu.SMEM),
        ],
        out_specs=pl.BlockSpec(memory_space=pltpu.VMEM),
        scratch_shapes=[
            pltpu.VMEM((NK, B), jnp.float32),
            pltpu.VMEM((B, HD), jnp.float32),
            pltpu.VMEM((B, HD), jnp.float32),
            pltpu.VMEM((1, BH), jnp.float32),
            pltpu.VMEM((1, BH), jnp.float32),
            pltpu.VMEM((1, BH), jnp.float32),
            pltpu.VMEM((1, BH), jnp.float32),
            pltpu.SemaphoreType.DMA((3,)),
            pltpu.SemaphoreType.DMA((3,)),
        ],
        compiler_params=pltpu.CompilerParams(collective_id=0),
    )(Qr, Kr, Vr, bt, lens)
    return out.reshape(B, 1, H, D)
